# baseline (speedup 1.0000x reference)
import jax
import jax.numpy as jnp
from jax import lax
from jax.experimental import pallas as pl
from jax.experimental.pallas import tpu as pltpu
from jax.experimental.pallas import tpu_sc as plsc

N = 10000
E = 320000
D = 128

NC = 2
NS = 16
B = 80
SUPERS = 4
CB = 32
CPT = SUPERS * CB
EROWS = NC * NS * CPT
EPAD = EROWS * B
ROWS_PER_TILE = 624
TAIL_ROWS = N - NS * ROWS_PER_TILE


def _sc_body(x_hbm, src_hbm, dst_hbm, attn_hbm, hp_hbm,
             acc, srcb, dstb, attnb, rows,
             sem_i0, sem_i1, sem_g0, sem_g1, sem_s0, sem_s1):
    sem_i = (sem_i0, sem_i1)
    sem_g = (sem_g0, sem_g1)
    sem_s = (sem_s0, sem_s1)
    c = lax.axis_index("c")
    s = lax.axis_index("s")
    t0 = (c * NS + s) * CPT

    def issue_idx(sup, bs):
        r0 = t0 + sup * CB
        pltpu.async_copy(src_hbm.at[pl.ds(r0, CB)], srcb.at[bs], sem_i[bs])
        pltpu.async_copy(dst_hbm.at[pl.ds(r0, CB)], dstb.at[bs], sem_i[bs])
        pltpu.async_copy(attn_hbm.at[pl.ds(r0, CB)], attnb.at[bs],
                         sem_i[bs])

    def wait_idx(sup, bs):
        r0 = t0 + sup * CB
        pltpu.make_async_copy(src_hbm.at[pl.ds(r0, CB)], srcb.at[bs],
                              sem_i[bs]).wait()
        pltpu.make_async_copy(dst_hbm.at[pl.ds(r0, CB)], dstb.at[bs],
                              sem_i[bs]).wait()
        pltpu.make_async_copy(attn_hbm.at[pl.ds(r0, CB)], attnb.at[bs],
                              sem_i[bs]).wait()

    issue_idx(0, 0)

    def zrow(i, carry):
        for j in range(D // 16):
            rows[0, i, pl.ds(j * 16, 16)] = jnp.zeros((16,), jnp.float32)
        return carry

    lax.fori_loop(0, B, zrow, 0)
    for q in range(ROWS_PER_TILE // B):
        pltpu.sync_copy(rows.at[0],
                        acc.at[pl.ds(s * ROWS_PER_TILE + q * B, B)])
    rem = ROWS_PER_TILE - (ROWS_PER_TILE // B) * B
    pltpu.sync_copy(
        rows.at[0, pl.ds(0, rem)],
        acc.at[pl.ds(s * ROWS_PER_TILE + ROWS_PER_TILE - rem, rem)])

    @pl.when(s == NS - 1)
    def _zero_tail():
        pltpu.sync_copy(rows.at[0, pl.ds(0, TAIL_ROWS)],
                        acc.at[pl.ds(NS * ROWS_PER_TILE, TAIL_ROWS)])

    plsc.subcore_barrier()

    def issue_gather(bs, j, b):
        pltpu.async_copy(x_hbm.at[srcb.at[bs, j]], rows.at[b], sem_g[b])

    def wait_gather(bs, j, b):
        pltpu.make_async_copy(x_hbm.at[srcb.at[bs, j]], rows.at[b],
                              sem_g[b]).wait()

    def issue_scatter(bs, j, b):
        pltpu.async_copy(rows.at[b], acc.at[dstb.at[bs, j]], sem_s[b],
                         add=True)

    def wait_scatter(bs, j, b):
        pltpu.make_async_copy(rows.at[b], acc.at[dstb.at[bs, j]],
                              sem_s[b]).wait()

    def scale(bs, j, b):
        def rowscale(g, rcarry):
            av = attnb[bs, j, pl.ds(g * 16, 16)]
            for t in range(16):
                a = jnp.full((16,), av[t], jnp.float32)
                for f in range(D // 16):
                    rows[b, g * 16 + t, pl.ds(f * 16, 16)] = (
                        rows[b, g * 16 + t, pl.ds(f * 16, 16)] * a)
            return rcarry

        lax.fori_loop(0, B // 16, rowscale, 0)

    for sup in range(SUPERS):
        bs = sup & 1
        wait_idx(sup, bs)
        if sup + 1 < SUPERS:
            issue_idx(sup + 1, 1 - bs)

        issue_gather(bs, 0, 0)
        issue_gather(bs, 1, 1)
        wait_gather(bs, 0, 0)
        scale(bs, 0, 0)
        issue_scatter(bs, 0, 0)

        def loop_body(k, carry, bs=bs):
            for off in range(2):
                ki = k + off
                b = 1 - off
                nb = off
                wait_scatter(bs, ki - 1, nb)
                issue_gather(bs, ki + 1, nb)
                wait_gather(bs, ki, b)
                scale(bs, ki, b)
                issue_scatter(bs, ki, b)
            return carry

        lax.fori_loop(0, (CB - 2) // 2,
                      lambda i, cy: loop_body(1 + 2 * i, cy), 0)
        wait_scatter(bs, CB - 2, 0)
        wait_gather(bs, CB - 1, 1)
        scale(bs, CB - 1, 1)
        issue_scatter(bs, CB - 1, 1)
        wait_scatter(bs, CB - 1, 1)

    plsc.subcore_barrier()

    pltpu.sync_copy(acc.at[pl.ds(s * ROWS_PER_TILE, ROWS_PER_TILE)],
                    hp_hbm.at[c, pl.ds(s * ROWS_PER_TILE, ROWS_PER_TILE)])

    @pl.when(s == NS - 1)
    def _drain_tail():
        pltpu.sync_copy(acc.at[pl.ds(NS * ROWS_PER_TILE, TAIL_ROWS)],
                        hp_hbm.at[c, pl.ds(NS * ROWS_PER_TILE, TAIL_ROWS)])


def _sc_message_passing(x, src2, dst2, attn2):
    mesh = plsc.VectorSubcoreMesh(core_axis_name="c", subcore_axis_name="s")
    kern = pl.kernel(
        _sc_body,
        mesh=mesh,
        out_type=jax.ShapeDtypeStruct((NC, N, D), jnp.float32),
        scratch_types=[
            pltpu.VMEM_SHARED((N, D), jnp.float32),
            pltpu.VMEM((2, CB, B), jnp.int32),
            pltpu.VMEM((2, CB, B), jnp.int32),
            pltpu.VMEM((2, CB, B), jnp.float32),
            pltpu.VMEM((2, B, D), jnp.float32),
            pltpu.SemaphoreType.DMA,
            pltpu.SemaphoreType.DMA,
            pltpu.SemaphoreType.DMA,
            pltpu.SemaphoreType.DMA,
            pltpu.SemaphoreType.DMA,
            pltpu.SemaphoreType.DMA,
        ],
    )
    return kern(x, src2, dst2, attn2)


def _tc_body(x_ref, h0_ref, h1_ref, w1_ref, b1_ref, w2_ref, b2_ref, o_ref):
    x = x_ref[...]
    hn = h0_ref[0] + h1_ref[0]
    u = x + hn
    v = x * hn
    dn = (((1,), (1,)), ((), ()))
    y1 = lax.dot_general(u, w1_ref[...], dn,
                         preferred_element_type=jnp.float32) + b1_ref[...]
    y1 = jnp.where(y1 >= 0, y1, y1 * 0.01)
    y2 = lax.dot_general(v, w2_ref[...], dn,
                         preferred_element_type=jnp.float32) + b2_ref[...]
    y2 = jnp.where(y2 >= 0, y2, y2 * 0.01)
    o_ref[...] = y1 + y2


def _tc_dense(x, hp, W1, b1, W2, b2):
    BN = 1000
    grid = (N // BN,)
    row_spec = pl.BlockSpec((BN, D), lambda i: (i, 0))
    h0_spec = pl.BlockSpec((1, BN, D), lambda i: (0, i, 0))
    h1_spec = pl.BlockSpec((1, BN, D), lambda i: (1, i, 0))
    full_spec = pl.BlockSpec((D, D), lambda i: (0, 0))
    bias_spec = pl.BlockSpec((1, D), lambda i: (0, 0))
    return pl.pallas_call(
        _tc_body,
        grid=grid,
        in_specs=[row_spec, h0_spec, h1_spec, full_spec, bias_spec,
                  full_spec, bias_spec],
        out_specs=row_spec,
        out_shape=jax.ShapeDtypeStruct((N, D), jnp.float32),
    )(x, hp, hp, W1, b1, W2, b2)


@jax.jit
def kernel(x, edge_index, edge_attn, W1, b1, W2, b2):
    pad = EPAD - E
    src2 = jnp.concatenate(
        [edge_index[0], jnp.zeros((pad,), jnp.int32)]).reshape(EROWS, B)
    dst2 = jnp.concatenate(
        [edge_index[1], jnp.zeros((pad,), jnp.int32)]).reshape(EROWS, B)
    attn2 = jnp.concatenate(
        [edge_attn.reshape(E), jnp.zeros((pad,), jnp.float32)]
    ).reshape(EROWS, B)
    hp = _sc_message_passing(x, src2, dst2, attn2)
    out = _tc_dense(x, hp, W1, b1.reshape(1, D), W2, b2.reshape(1, D))
    return out
